# baseline (speedup 1.0000x reference)
import math
from functools import lru_cache, partial

import numpy as np
import jax
import jax.numpy as jnp
from jax.experimental import pallas as pl
from jax.experimental.pallas import tpu as pltpu

_VMEM_LIMIT_BYTES = 48 * 1024 * 1024


def _nearest_indices(in_dim: int, out_dim: int) -> np.ndarray:
    src = np.floor(np.arange(out_dim, dtype=np.float32) * np.float32(in_dim / out_dim))
    return np.clip(src.astype(np.int64), 0, in_dim - 1)


@lru_cache(maxsize=16)
def _sel_pair_mats(w_in: int, w_out: int, q: int):
    idx = _nearest_indices(w_in, w_out)
    mats = []
    for p in range(q):
        m = np.zeros((q * w_in, w_out), dtype=np.float32)
        m[p * w_in + idx, np.arange(w_out)] = 1.0
        mats.append(jnp.asarray(m))
    return tuple(mats)


def _upsample_pairs_kernel(*refs, sf_h, q):
    sel_refs, x_ref, o_ref = refs[:q], refs[q], refs[q + 1]
    u = x_ref[...]
    stride = q * sf_h
    for p in range(q):
        t = jnp.dot(u, sel_refs[p][...], preferred_element_type=jnp.float32)
        for j in range(sf_h):
            o_ref[p * sf_h + j::stride, :] = t


def kernel(x):
    N, C, H_in, W_in = x.shape
    sf_h = sf_w = 2
    H_out, W_out = H_in * sf_h, W_in * sf_w

    orig_dtype = x.dtype
    if not jnp.issubdtype(x.dtype, jnp.floating):
        x = x.astype(jnp.float32)

    NC = N * C
    q = 128 // W_in if (128 % W_in == 0 and W_in < 128) else 1
    while q > 1 and H_in % q:
        q //= 2

    c_blk = 128
    while NC % c_blk:
        c_blk //= 2
    grid = NC // c_blk

    sels = _sel_pair_mats(W_in, W_out, q)
    sels = tuple(s.astype(x.dtype) for s in sels)
    xp = x.reshape(NC * H_in // q, q * W_in)

    out2d = pl.pallas_call(
        partial(_upsample_pairs_kernel, sf_h=sf_h, q=q),
        out_shape=jax.ShapeDtypeStruct((NC * H_out, W_out), x.dtype),
        grid=(grid,),
        in_specs=[pl.BlockSpec((q * W_in, W_out), lambda i: (0, 0))] * q
        + [pl.BlockSpec((c_blk * H_in // q, q * W_in), lambda i: (i, 0))],
        out_specs=pl.BlockSpec((c_blk * H_out, W_out), lambda i: (i, 0)),
        compiler_params=pltpu.CompilerParams(
            dimension_semantics=("parallel",),
            vmem_limit_bytes=_VMEM_LIMIT_BYTES,
        ),
    )(*sels, xp)

    out = out2d.reshape(N, C, H_out, W_out)
    if out.dtype != orig_dtype:
        out = out.astype(orig_dtype)
    return out


# speedup vs baseline: 2.6640x; 2.6640x over previous
import math
from functools import lru_cache, partial

import numpy as np
import jax
import jax.numpy as jnp
from jax.experimental import pallas as pl
from jax.experimental.pallas import tpu as pltpu

_VMEM_LIMIT_BYTES = 48 * 1024 * 1024


def _nearest_indices(in_dim: int, out_dim: int) -> np.ndarray:
    src = np.floor(np.arange(out_dim, dtype=np.float32) * np.float32(in_dim / out_dim))
    return np.clip(src.astype(np.int64), 0, in_dim - 1)


@lru_cache(maxsize=16)
def _sel_w_mat(w_in: int, w_out: int):
    idx = _nearest_indices(w_in, w_out)
    m = np.zeros((w_in, w_out), dtype=np.float32)
    m[idx, np.arange(w_out)] = 1.0
    return jnp.asarray(m)


def _upsample_kernel(sel_w_ref, x_ref, o_ref, *, sf_h):
    h_in, w_in, c = x_ref.shape[1], x_ref.shape[2], x_ref.shape[3]
    v = jnp.transpose(x_ref[0], (2, 0, 1))
    v2 = v.reshape(c * h_in, w_in)
    t = jnp.dot(v2, sel_w_ref[...], preferred_element_type=jnp.float32)
    for j in range(sf_h):
        o_ref[j::sf_h, :] = t


def kernel(x):
    N, C, H_in, W_in = x.shape
    sf_h = sf_w = 2
    H_out, W_out = H_in * sf_h, W_in * sf_w

    orig_dtype = x.dtype
    if not jnp.issubdtype(x.dtype, jnp.floating):
        x = x.astype(jnp.float32)

    sel_w = _sel_w_mat(W_in, W_out).astype(x.dtype)
    x_nhwc = jnp.transpose(x, (0, 2, 3, 1))

    out2d = pl.pallas_call(
        partial(_upsample_kernel, sf_h=sf_h),
        out_shape=jax.ShapeDtypeStruct((N * C * H_out, W_out), x.dtype),
        grid=(N,),
        in_specs=[
            pl.BlockSpec((W_in, W_out), lambda n: (0, 0)),
            pl.BlockSpec((1, H_in, W_in, C), lambda n: (n, 0, 0, 0)),
        ],
        out_specs=pl.BlockSpec((C * H_out, W_out), lambda n: (n, 0)),
        compiler_params=pltpu.CompilerParams(
            dimension_semantics=("parallel",),
            vmem_limit_bytes=_VMEM_LIMIT_BYTES,
        ),
    )(sel_w, x_nhwc)

    out = out2d.reshape(N, C, H_out, W_out)
    if out.dtype != orig_dtype:
        out = out.astype(orig_dtype)
    return out


# speedup vs baseline: 2.7693x; 1.0395x over previous
import math
from functools import lru_cache, partial

import numpy as np
import jax
import jax.numpy as jnp
from jax.experimental import pallas as pl
from jax.experimental.pallas import tpu as pltpu

_VMEM_LIMIT_BYTES = 48 * 1024 * 1024


def _nearest_indices(in_dim: int, out_dim: int) -> np.ndarray:
    src = np.floor(np.arange(out_dim, dtype=np.float32) * np.float32(in_dim / out_dim))
    return np.clip(src.astype(np.int64), 0, in_dim - 1)


@lru_cache(maxsize=16)
def _sel_w_mat(w_in: int, w_out: int):
    idx = _nearest_indices(w_in, w_out)
    m = np.zeros((w_in, w_out), dtype=np.float32)
    m[idx, np.arange(w_out)] = 1.0
    return jnp.asarray(m)


def _upsample_kernel(sel_w_ref, x_ref, o_ref, *, sf_h):
    nb, h_in, w_in, c = x_ref.shape
    rows = c * h_in
    for b in range(nb):
        v = jnp.transpose(x_ref[b], (2, 0, 1))
        v2 = v.reshape(rows, w_in)
        t = jnp.dot(v2, sel_w_ref[...], preferred_element_type=jnp.float32)
        for j in range(sf_h):
            o_ref[b * sf_h * rows + j:(b + 1) * sf_h * rows:sf_h, :] = t


def kernel(x):
    N, C, H_in, W_in = x.shape
    sf_h = sf_w = 2
    H_out, W_out = H_in * sf_h, W_in * sf_w

    orig_dtype = x.dtype
    if not jnp.issubdtype(x.dtype, jnp.floating):
        x = x.astype(jnp.float32)

    sel_w = _sel_w_mat(W_in, W_out).astype(x.dtype)
    x_nhwc = jnp.transpose(x, (0, 2, 3, 1))

    nb = 2 if N % 2 == 0 else 1
    out2d = pl.pallas_call(
        partial(_upsample_kernel, sf_h=sf_h),
        out_shape=jax.ShapeDtypeStruct((N * C * H_out, W_out), x.dtype),
        grid=(N // nb,),
        in_specs=[
            pl.BlockSpec((W_in, W_out), lambda n: (0, 0)),
            pl.BlockSpec((nb, H_in, W_in, C), lambda n: (n, 0, 0, 0)),
        ],
        out_specs=pl.BlockSpec((nb * C * H_out, W_out), lambda n: (n, 0)),
        compiler_params=pltpu.CompilerParams(
            dimension_semantics=("parallel",),
            vmem_limit_bytes=58 * 1024 * 1024,
        ),
    )(sel_w, x_nhwc)

    out = out2d.reshape(N, C, H_out, W_out)
    if out.dtype != orig_dtype:
        out = out.astype(orig_dtype)
    return out
